# baseline (speedup 1.0000x reference)
import functools

import jax
import jax.numpy as jnp
from jax import lax
from jax.experimental import pallas as pl
from jax.experimental.pallas import tpu as pltpu
from jax.experimental.pallas import tpu_sc as plsc

K = 100
ROWS = 128
N = 32768
NV = N // 16
U = 8
CU = 16
SAMPLE_KK = 32
OUTP = 112
ROWS_PER_W = ROWS // 32

_MESH = plsc.VectorSubcoreMesh(core_axis_name="c", subcore_axis_name="s")


def _convert(x):
    b = plsc.bitcast(x, jnp.int32)
    m = lax.shift_right_arithmetic(b, 31)
    y = lax.bitwise_xor(b, lax.bitwise_or(m, jnp.int32(-2147483648)))
    return plsc.bitcast(y, jnp.uint32)


def _unconvert(y):
    s = lax.shift_right_arithmetic(plsc.bitcast(y, jnp.int32), 31)
    mask = lax.bitwise_or(jnp.int32(-2147483648), lax.bitwise_not(s))
    b = lax.bitwise_xor(plsc.bitcast(y, jnp.int32), mask)
    return plsc.bitcast(b, jnp.float32)


def _sort16d(x):
    r = plsc.sort_key_val(x, x, descending=True)
    if isinstance(r, (tuple, list)):
        return r[0]
    return r


def _clean(c):
    n = len(c)
    if n == 1:
        return [_sort16d(c[0])]
    h = n // 2
    hi = [jnp.maximum(c[i], c[i + h]) for i in range(h)]
    lo = [jnp.minimum(c[i], c[i + h]) for i in range(h)]
    return _clean(hi) + _clean(lo)


def _merge(a, b):
    n = len(a)
    br = [jnp.flip(x) for x in reversed(b)]
    hi = [jnp.maximum(a[i], br[i]) for i in range(n)]
    lo = [jnp.minimum(a[i], br[i]) for i in range(n)]
    return _clean(hi) + _clean(lo)


def _sort128d(v):
    v = [_sort16d(x) for x in v]
    v = [_merge([v[2 * i]], [v[2 * i + 1]]) for i in range(4)]
    v = [_merge(v[0], v[1]), _merge(v[2], v[3])]
    v = _merge(v[0], v[1])
    return v


def _body(cands_hbm, out_hbm, inbuf, candbuf, hist, fin, outbuf, s0, s1, s2):
    lane = jnp.arange(16, dtype=jnp.int32)
    ones16 = jnp.ones((16,), jnp.int32)
    zeros16 = jnp.zeros((16,), jnp.int32)
    sems = [s0, s1]

    wid = lax.axis_index("s") * 2 + lax.axis_index("c")
    row0 = wid * ROWS_PER_W

    def clear_hist():
        def clr(j, _):
            for u in range(U):
                hist[pl.ds((j * U + u) * 16, 16)] = zeros16
            return 0

        lax.fori_loop(0, 256 // U, clr, 0)

    def digit_scan(kk):
        def gcond(c):
            g, cum = c
            return jnp.logical_and(cum < kk, g >= 0)

        def gbody(c):
            g, cum = c
            acc = hist[pl.ds(g * 256, 16)]
            for t in range(1, 16):
                acc = acc + hist[pl.ds(g * 256 + t * 16, 16)]
            return (g - 1, cum + jnp.sum(acc))

        g, cum_g = lax.while_loop(gcond, gbody, (jnp.int32(15), jnp.int32(0)))
        g_sel = g + 1
        acc = hist[pl.ds(g_sel * 256, 16)]
        for t in range(1, 16):
            acc = acc + hist[pl.ds(g_sel * 256 + t * 16, 16)]
        cum_before = cum_g - jnp.sum(acc)

        def dcond(c):
            d, cum = c
            return cum < kk

        def dbody(c):
            d, cum = c
            return (d - 1, cum + jnp.sum(hist[pl.ds(d * 16, 16)]))

        d, cum = lax.while_loop(
            dcond, dbody, (g_sel * 16 + 15, cum_before))
        d_sel = d + 1
        t_sel = jnp.sum(hist[pl.ds(d_sel * 16, 16)])
        return d_sel, cum, t_sel

    def process(cur, ob, row):
        def compact_row(thresh):
            tvec = _unconvert(jnp.broadcast_to(thresh, (16,)))
            tvec = jnp.where(thresh == 0,
                             jnp.broadcast_to(jnp.float32(-jnp.inf), (16,)),
                             tvec)

            def c1(j, cnt):
                for u in range(CU):
                    x = inbuf[cur, pl.ds((j * CU + u) * 16, 16)]
                    keep = x >= tvec
                    plsc.store_scatter(
                        candbuf, [cnt * 16 + lane], plsc.bitcast(x, jnp.int32),
                        mask=keep)
                    cnt = cnt + keep.astype(jnp.int32)
                return cnt

            return lax.fori_loop(0, NV // CU, c1, zeros16)

        scope = jax.named_scope
        with scope("clear1"):
            clear_hist()

        def ps(j, _):
            for u in range(8):
                y = _convert(inbuf[cur, pl.ds((j * 8 + u) * 8 * 16, 16)])
                d = lax.shift_right_logical(y, jnp.uint32(24))
                plsc.addupdate_scatter(
                    hist, [d.astype(jnp.int32) * 16 + lane], ones16)
            return 0

        with scope("sample_hist"):
            lax.fori_loop(0, NV // 8 // 8, ps, 0)
        with scope("sample_scan"):
            d_hat, _, _ = digit_scan(jnp.int32(SAMPLE_KK))
        l_hat = lax.shift_left(d_hat.astype(jnp.uint32), jnp.uint32(24))

        with scope("compact_full"):
            cnt = compact_row(l_hat)
        m = jnp.sum(cnt)

        def fallback(_):
            clear_hist()

            def pa(j, __):
                for u in range(U):
                    y = _convert(inbuf[cur, pl.ds((j * U + u) * 16, 16)])
                    d = lax.shift_right_logical(y, jnp.uint32(24))
                    plsc.addupdate_scatter(
                        hist, [d.astype(jnp.int32) * 16 + lane], ones16)
                return 0

            lax.fori_loop(0, NV // U, pa, 0)
            d1, _, __ = digit_scan(jnp.int32(K))
            l1 = lax.shift_left(d1.astype(jnp.uint32), jnp.uint32(24))
            return compact_row(l1)

        with scope("fallback_cond"):
            cnt = lax.cond(m < K, fallback, lambda _: cnt, 0)

        maxc = jnp.max(cnt)
        with scope("clear2"):
            clear_hist()

        def lh(j, _):
            y = _convert(plsc.bitcast(candbuf[pl.ds(j * 16, 16)], jnp.float32))
            d = lax.shift_right_logical(y, jnp.uint32(24))
            plsc.addupdate_scatter(
                hist, [d.astype(jnp.int32) * 16 + lane], ones16, mask=j < cnt)
            return 0

        with scope("list_hist"):
            lax.fori_loop(0, maxc, lh, 0)
        with scope("list_scan"):
            d1, cum1, t1 = digit_scan(jnp.int32(K))
        l_cur = lax.shift_left(d1.astype(jnp.uint32), jnp.uint32(24))
        hi = cum1 - t1
        m = cum1

        def lc(j, c2):
            y = _convert(plsc.bitcast(candbuf[pl.ds(j * 16, 16)], jnp.float32))
            keep = jnp.logical_and(j < cnt, y >= l_cur)
            plsc.store_scatter(
                candbuf, [c2 * 16 + lane], plsc.bitcast(y, jnp.int32),
                mask=keep)
            return c2 + keep.astype(jnp.int32)

        with scope("list_compact"):
            cnt = lax.fori_loop(0, maxc, lc, zeros16)

        def lvl_cond(c):
            l_c, hi_c, m_c, sh, cnt_c = c
            return jnp.logical_and(m_c > 128, sh >= 0)

        def lvl_body(c):
            l_c, hi_c, m_c, sh, cnt_c = c
            sh_u = sh.astype(jnp.uint32)
            psh_u = (sh + 8).astype(jnp.uint32)
            lp = lax.shift_right_logical(l_c, psh_u)
            maxc2 = jnp.max(cnt_c)
            clear_hist()

            def hb(j, _):
                y = plsc.bitcast(candbuf[pl.ds(j * 16, 16)], jnp.uint32)
                inb = jnp.logical_and(
                    j < cnt_c, lax.shift_right_logical(y, psh_u) == lp)
                dg = lax.bitwise_and(
                    lax.shift_right_logical(y, sh_u), jnp.uint32(255))
                plsc.addupdate_scatter(
                    hist, [dg.astype(jnp.int32) * 16 + lane], ones16, mask=inb)
                return 0

            lax.fori_loop(0, maxc2, hb, 0)

            d2, cum2, t2 = digit_scan(K - hi_c)
            new_l = l_c + lax.shift_left(d2.astype(jnp.uint32), sh_u)
            new_hi = hi_c + cum2 - t2
            new_m = hi_c + cum2

            def cb(j, cnt2):
                yi = candbuf[pl.ds(j * 16, 16)]
                y = plsc.bitcast(yi, jnp.uint32)
                keep = jnp.logical_and(j < cnt_c, y >= new_l)
                plsc.store_scatter(candbuf, [cnt2 * 16 + lane], yi, mask=keep)
                return cnt2 + keep.astype(jnp.int32)

            new_cnt = lax.fori_loop(0, maxc2, cb, zeros16)
            return (new_l, new_hi, new_m, sh - 8, new_cnt)

        with scope("narrow_levels"):
            l_cur, hi, m, _, cnt = lax.while_loop(
                lvl_cond, lvl_body, (l_cur, hi, m, jnp.int32(16), cnt))

        small = m <= 128
        thr = l_cur - small.astype(jnp.uint32)
        padv = jnp.where(small, jnp.uint32(0), l_cur)
        pad16 = plsc.bitcast(jnp.broadcast_to(padv, (16,)), jnp.int32)
        for t in range(8):
            fin[pl.ds(t * 16, 16)] = pad16

        maxc3 = jnp.max(cnt)
        scope2 = jax.named_scope("final_gather")
        scope2.__enter__()

        def f1(j, kc):
            y = plsc.bitcast(candbuf[pl.ds(j * 16, 16)], jnp.uint32)
            keep = jnp.logical_and(j < cnt, y > thr)
            return kc + keep.astype(jnp.int32)

        kc = lax.fori_loop(0, maxc3, f1, zeros16)
        base = plsc.cumsum(kc) - kc

        def f2(j, rc):
            yi = candbuf[pl.ds(j * 16, 16)]
            y = plsc.bitcast(yi, jnp.uint32)
            keep = jnp.logical_and(j < cnt, y > thr)
            plsc.store_scatter(fin, [base + rc], yi, mask=keep)
            return rc + keep.astype(jnp.int32)

        lax.fori_loop(0, maxc3, f2, zeros16)
        scope2.__exit__(None, None, None)
        scope3 = jax.named_scope("sort_out")
        scope3.__enter__()
        v = _sort128d(
            [plsc.bitcast(fin[pl.ds(t * 16, 16)], jnp.uint32)
             for t in range(8)])
        for t in range(OUTP // 16):
            outbuf[ob, pl.ds(t * 16, 16)] = _unconvert(v[t])
        cpo = pltpu.async_copy(outbuf.at[ob], out_hbm.at[row], s2)
        scope3.__exit__(None, None, None)
        return cpo

    cp = pltpu.async_copy(cands_hbm.at[row0], inbuf.at[0], sems[0])
    outcps = []
    for i in range(ROWS_PER_W):
        nxt = None
        if i + 1 < ROWS_PER_W:
            nxt = pltpu.async_copy(
                cands_hbm.at[row0 + i + 1], inbuf.at[(i + 1) % 2],
                sems[(i + 1) % 2])
        cp.wait()
        outcps.append(process(i % 2, i, row0 + i))
        cp = nxt
    for c in outcps:
        c.wait()


_topk_sc = functools.partial(
    pl.kernel,
    out_type=jax.ShapeDtypeStruct((ROWS, OUTP), jnp.float32),
    mesh=_MESH,
    compiler_params=pltpu.CompilerParams(needs_layout_passes=False),
    scratch_types=[
        pltpu.VMEM((2, N), jnp.float32),
        pltpu.VMEM((N,), jnp.int32),
        pltpu.VMEM((4096,), jnp.int32),
        pltpu.VMEM((128,), jnp.int32),
        pltpu.VMEM((ROWS_PER_W, OUTP), jnp.float32),
        pltpu.SemaphoreType.DMA,
        pltpu.SemaphoreType.DMA,
        pltpu.SemaphoreType.DMA,
    ],
)(_body)


@jax.jit
def kernel(cands):
    return _topk_sc(cands)[:, :K]


# speedup vs baseline: 1.7866x; 1.7866x over previous
import functools

import jax
import jax.numpy as jnp
from jax import lax
from jax.experimental import pallas as pl
from jax.experimental.pallas import tpu as pltpu
from jax.experimental.pallas import tpu_sc as plsc

K = 100
ROWS = 128
N = 32768
NV = N // 16
U = 8
CU = 16
SAMPLE_KK = 32
OUTP = 112
ROWS_PER_W = ROWS // 32

_MESH = plsc.VectorSubcoreMesh(core_axis_name="c", subcore_axis_name="s")


def _convert(x):
    b = plsc.bitcast(x, jnp.int32)
    m = lax.shift_right_arithmetic(b, 31)
    y = lax.bitwise_xor(b, lax.bitwise_or(m, jnp.int32(-2147483648)))
    return plsc.bitcast(y, jnp.uint32)


def _unconvert(y):
    s = lax.shift_right_arithmetic(plsc.bitcast(y, jnp.int32), 31)
    mask = lax.bitwise_or(jnp.int32(-2147483648), lax.bitwise_not(s))
    b = lax.bitwise_xor(plsc.bitcast(y, jnp.int32), mask)
    return plsc.bitcast(b, jnp.float32)


def _sort16d(x):
    r = plsc.sort_key_val(x, x, descending=True)
    if isinstance(r, (tuple, list)):
        return r[0]
    return r


def _clean(c):
    n = len(c)
    if n == 1:
        return [_sort16d(c[0])]
    h = n // 2
    hi = [jnp.maximum(c[i], c[i + h]) for i in range(h)]
    lo = [jnp.minimum(c[i], c[i + h]) for i in range(h)]
    return _clean(hi) + _clean(lo)


def _merge(a, b):
    n = len(a)
    br = [jnp.flip(x) for x in reversed(b)]
    hi = [jnp.maximum(a[i], br[i]) for i in range(n)]
    lo = [jnp.minimum(a[i], br[i]) for i in range(n)]
    return _clean(hi) + _clean(lo)


def _sort128d(v):
    v = [_sort16d(x) for x in v]
    v = [_merge([v[2 * i]], [v[2 * i + 1]]) for i in range(4)]
    v = [_merge(v[0], v[1]), _merge(v[2], v[3])]
    v = _merge(v[0], v[1])
    return v


def _body(cands_hbm, out_hbm, inbuf, candbuf, hist, fin, outbuf, s0, s1, s2):
    lane = jnp.arange(16, dtype=jnp.int32)
    ones16 = jnp.ones((16,), jnp.int32)
    zeros16 = jnp.zeros((16,), jnp.int32)
    sems = [s0, s1]

    wid = lax.axis_index("s") * 2 + lax.axis_index("c")
    row0 = wid * ROWS_PER_W

    def clear_hist():
        def clr(j, _):
            for u in range(U):
                hist[pl.ds((j * U + u) * 16, 16)] = zeros16
            return 0

        lax.fori_loop(0, 256 // U, clr, 0)

    def digit_scan(kk):
        def gcond(c):
            g, cum = c
            return jnp.logical_and(cum < kk, g >= 0)

        def gbody(c):
            g, cum = c
            acc = hist[pl.ds(g * 256, 16)]
            for t in range(1, 16):
                acc = acc + hist[pl.ds(g * 256 + t * 16, 16)]
            return (g - 1, cum + jnp.sum(acc))

        g, cum_g = lax.while_loop(gcond, gbody, (jnp.int32(15), jnp.int32(0)))
        g_sel = g + 1
        acc = hist[pl.ds(g_sel * 256, 16)]
        for t in range(1, 16):
            acc = acc + hist[pl.ds(g_sel * 256 + t * 16, 16)]
        cum_before = cum_g - jnp.sum(acc)

        def dcond(c):
            d, cum = c
            return cum < kk

        def dbody(c):
            d, cum = c
            return (d - 1, cum + jnp.sum(hist[pl.ds(d * 16, 16)]))

        d, cum = lax.while_loop(
            dcond, dbody, (g_sel * 16 + 15, cum_before))
        d_sel = d + 1
        t_sel = jnp.sum(hist[pl.ds(d_sel * 16, 16)])
        return d_sel, cum, t_sel

    def process(cur, ob, row):
        def compact_row(thresh):
            tvec = _unconvert(jnp.broadcast_to(thresh, (16,)))
            tvec = jnp.where(thresh == 0,
                             jnp.broadcast_to(jnp.float32(-jnp.inf), (16,)),
                             tvec)

            def c1(j, cnt):
                xs = [inbuf[cur, pl.ds((j * CU + u) * 16, 16)]
                      for u in range(CU)]
                keeps = [x >= tvec for x in xs]
                for u in range(CU):
                    plsc.store_scatter(
                        candbuf, [cnt * 16 + lane],
                        plsc.bitcast(xs[u], jnp.int32), mask=keeps[u])
                    cnt = cnt + keeps[u].astype(jnp.int32)
                return cnt

            return lax.fori_loop(0, NV // CU, c1, zeros16)

        scope = jax.named_scope
        with scope("clear1"):
            clear_hist()

        def ps(j, _):
            ys = [_convert(inbuf[cur, pl.ds((j * 8 + u) * 8 * 16, 16)])
                  for u in range(8)]
            for y in ys:
                d = lax.shift_right_logical(y, jnp.uint32(24))
                plsc.addupdate_scatter(
                    hist, [d.astype(jnp.int32) * 16 + lane], ones16)
            return 0

        with scope("sample_hist"):
            lax.fori_loop(0, NV // 8 // 8, ps, 0)
        with scope("sample_scan"):
            d_hat, _, _ = digit_scan(jnp.int32(SAMPLE_KK))
        l_hat = lax.shift_left(d_hat.astype(jnp.uint32), jnp.uint32(24))

        with scope("compact_full"):
            cnt = compact_row(l_hat)
        m = jnp.sum(cnt)

        def fallback(_):
            clear_hist()

            def pa(j, __):
                ys = [_convert(inbuf[cur, pl.ds((j * U + u) * 16, 16)])
                      for u in range(U)]
                for y in ys:
                    d = lax.shift_right_logical(y, jnp.uint32(24))
                    plsc.addupdate_scatter(
                        hist, [d.astype(jnp.int32) * 16 + lane], ones16)
                return 0

            lax.fori_loop(0, NV // U, pa, 0)
            d1, _, __ = digit_scan(jnp.int32(K))
            l1 = lax.shift_left(d1.astype(jnp.uint32), jnp.uint32(24))
            return compact_row(l1)

        with scope("fallback_cond"):
            cnt = lax.cond(m < K, fallback, lambda _: cnt, 0)

        maxc = jnp.max(cnt)
        with scope("clear2"):
            clear_hist()

        def lh(j, _):
            y = _convert(plsc.bitcast(candbuf[pl.ds(j * 16, 16)], jnp.float32))
            d = lax.shift_right_logical(y, jnp.uint32(24))
            plsc.addupdate_scatter(
                hist, [d.astype(jnp.int32) * 16 + lane], ones16, mask=j < cnt)
            return 0

        with scope("list_hist"):
            lax.fori_loop(0, maxc, lh, 0)
        with scope("list_scan"):
            d1, cum1, t1 = digit_scan(jnp.int32(K))
        l_cur = lax.shift_left(d1.astype(jnp.uint32), jnp.uint32(24))
        hi = cum1 - t1
        m = cum1

        def lc(j, c2):
            y = _convert(plsc.bitcast(candbuf[pl.ds(j * 16, 16)], jnp.float32))
            keep = jnp.logical_and(j < cnt, y >= l_cur)
            plsc.store_scatter(
                candbuf, [c2 * 16 + lane], plsc.bitcast(y, jnp.int32),
                mask=keep)
            return c2 + keep.astype(jnp.int32)

        with scope("list_compact"):
            cnt = lax.fori_loop(0, maxc, lc, zeros16)

        def lvl_cond(c):
            l_c, hi_c, m_c, sh, cnt_c = c
            return jnp.logical_and(m_c > 128, sh >= 0)

        def lvl_body(c):
            l_c, hi_c, m_c, sh, cnt_c = c
            sh_u = sh.astype(jnp.uint32)
            psh_u = (sh + 8).astype(jnp.uint32)
            lp = lax.shift_right_logical(l_c, psh_u)
            maxc2 = jnp.max(cnt_c)
            clear_hist()

            def hb(j, _):
                y = plsc.bitcast(candbuf[pl.ds(j * 16, 16)], jnp.uint32)
                inb = jnp.logical_and(
                    j < cnt_c, lax.shift_right_logical(y, psh_u) == lp)
                dg = lax.bitwise_and(
                    lax.shift_right_logical(y, sh_u), jnp.uint32(255))
                plsc.addupdate_scatter(
                    hist, [dg.astype(jnp.int32) * 16 + lane], ones16, mask=inb)
                return 0

            lax.fori_loop(0, maxc2, hb, 0)

            d2, cum2, t2 = digit_scan(K - hi_c)
            new_l = l_c + lax.shift_left(d2.astype(jnp.uint32), sh_u)
            new_hi = hi_c + cum2 - t2
            new_m = hi_c + cum2

            def cb(j, cnt2):
                yi = candbuf[pl.ds(j * 16, 16)]
                y = plsc.bitcast(yi, jnp.uint32)
                keep = jnp.logical_and(j < cnt_c, y >= new_l)
                plsc.store_scatter(candbuf, [cnt2 * 16 + lane], yi, mask=keep)
                return cnt2 + keep.astype(jnp.int32)

            new_cnt = lax.fori_loop(0, maxc2, cb, zeros16)
            return (new_l, new_hi, new_m, sh - 8, new_cnt)

        with scope("narrow_levels"):
            l_cur, hi, m, _, cnt = lax.while_loop(
                lvl_cond, lvl_body, (l_cur, hi, m, jnp.int32(16), cnt))

        small = m <= 128
        thr = l_cur - small.astype(jnp.uint32)
        padv = jnp.where(small, jnp.uint32(0), l_cur)
        pad16 = plsc.bitcast(jnp.broadcast_to(padv, (16,)), jnp.int32)
        for t in range(8):
            fin[pl.ds(t * 16, 16)] = pad16

        maxc3 = jnp.max(cnt)
        scope2 = jax.named_scope("final_gather")
        scope2.__enter__()

        def f1(j, kc):
            y = plsc.bitcast(candbuf[pl.ds(j * 16, 16)], jnp.uint32)
            keep = jnp.logical_and(j < cnt, y > thr)
            return kc + keep.astype(jnp.int32)

        kc = lax.fori_loop(0, maxc3, f1, zeros16)
        base = plsc.cumsum(kc) - kc

        def f2(j, rc):
            yi = candbuf[pl.ds(j * 16, 16)]
            y = plsc.bitcast(yi, jnp.uint32)
            keep = jnp.logical_and(j < cnt, y > thr)
            plsc.store_scatter(fin, [base + rc], yi, mask=keep)
            return rc + keep.astype(jnp.int32)

        lax.fori_loop(0, maxc3, f2, zeros16)
        scope2.__exit__(None, None, None)
        scope3 = jax.named_scope("sort_out")
        scope3.__enter__()
        v = _sort128d(
            [plsc.bitcast(fin[pl.ds(t * 16, 16)], jnp.uint32)
             for t in range(8)])
        for t in range(OUTP // 16):
            outbuf[ob, pl.ds(t * 16, 16)] = _unconvert(v[t])
        cpo = pltpu.async_copy(outbuf.at[ob], out_hbm.at[row], s2)
        scope3.__exit__(None, None, None)
        return cpo

    cp = pltpu.async_copy(cands_hbm.at[row0], inbuf.at[0], sems[0])
    outcps = []
    for i in range(ROWS_PER_W):
        nxt = None
        if i + 1 < ROWS_PER_W:
            nxt = pltpu.async_copy(
                cands_hbm.at[row0 + i + 1], inbuf.at[(i + 1) % 2],
                sems[(i + 1) % 2])
        cp.wait()
        outcps.append(process(i % 2, i, row0 + i))
        cp = nxt
    for c in outcps:
        c.wait()


_topk_sc = functools.partial(
    pl.kernel,
    out_type=jax.ShapeDtypeStruct((ROWS, OUTP), jnp.float32),
    mesh=_MESH,
    compiler_params=pltpu.CompilerParams(needs_layout_passes=False),
    scratch_types=[
        pltpu.VMEM((2, N), jnp.float32),
        pltpu.VMEM((N,), jnp.int32),
        pltpu.VMEM((4096,), jnp.int32),
        pltpu.VMEM((128,), jnp.int32),
        pltpu.VMEM((ROWS_PER_W, OUTP), jnp.float32),
        pltpu.SemaphoreType.DMA,
        pltpu.SemaphoreType.DMA,
        pltpu.SemaphoreType.DMA,
    ],
)(_body)


@jax.jit
def kernel(cands):
    return _topk_sc(cands)[:, :K]
